# baseline (speedup 1.0000x reference)
import dataclasses
import functools
import math

import jax
import jax.numpy as jnp
from jax import lax
from jax.experimental import pallas as pl
from jax.experimental.pallas import tpu as pltpu
from jax.experimental.pallas import tpu_sc as plsc

ALPHA = 0.99
LOG_ALPHA = math.log(ALPHA)

M = 1000000
D = 32
B = 16384

NS = 16
BPT = B // NS
NROW = 8
NDUM = 64
TPAD = M + NDUM * 16


def _sc_body(i_hbm, x2_hbm, c2_hbm, n_hbm,
             old_o, xw_o, cnt_o, w_o,
             idx2d, val2d, msk2d, iq2d, t_v, cntf_v,
             stag0, stag1,
             myflag_v, flags_v, table, flags_sp, sem, sem2):
    core = lax.axis_index("c")
    sub = lax.axis_index("s")

    @pl.when(core == 0)
    def _():
        base = sub * BPT

        pltpu.sync_copy(i_hbm.at[pl.ds(sub * NROW, NROW)], idx2d)

        cnt_cps = [pltpu.async_copy(
            n_hbm.at[idx2d.at[j]],
            cntf_v.at[pl.ds(j * 128, 128)], sem2) for j in range(NROW)]

        for r in range(NROW):
            for c in range(8):
                sl = pl.ds(c * 16, 16)
                iq2d[r, sl] = lax.shift_right_logical(idx2d[r, sl], 2)

        stags = [stag0, stag1]
        c_cps = [pltpu.async_copy(c2_hbm.at[iq2d.at[0]], stags[0], sem2)]

        for r in range(NROW):
            rbase = base + r * 128
            for c in range(8):
                val2d[r, pl.ds(c * 16, 16)] = (
                    rbase + c * 16 + lax.iota(jnp.int32, 16))

        cps = [pltpu.async_copy(val2d.at[j], table.at[idx2d.at[j]], sem)
               for j in range(NROW)]
        for cp in cps:
            cp.wait()
        plsc.subcore_barrier()

        def gather_t():
            cps = [pltpu.async_copy(table.at[idx2d.at[j]],
                                    t_v.at[pl.ds(j * 128, 128)], sem)
                   for j in range(NROW)]
            for cp in cps:
                cp.wait()

        def compute_pending():
            acc = jnp.zeros((16,), jnp.int32)
            for r in range(NROW):
                racc = jnp.zeros((16,), jnp.int32)
                for c in range(8):
                    sl = pl.ds(c * 16, 16)
                    t16 = t_v[pl.ds(r * 128 + c * 16, 16)]
                    v16 = val2d[r, sl]
                    i16 = idx2d[r, sl]
                    pend = t16 < v16
                    grp = (sub * NROW * 8 + r * 8 + c) % NDUM
                    dummy = M + grp * 16 + lax.iota(jnp.int32, 16)
                    msk2d[r, sl] = jnp.where(pend, i16, dummy)
                    racc = racc + jnp.where(pend, 1, 0).astype(jnp.int32)
                acc = acc + racc
            return jnp.sum(acc)

        def publish(total):
            myflag_v[...] = jnp.full((16,), total, jnp.int32)
            pltpu.sync_copy(myflag_v, flags_sp.at[sub])
            plsc.subcore_barrier()
            pltpu.sync_copy(flags_sp, flags_v)
            plsc.subcore_barrier()
            gacc = jnp.zeros((16,), jnp.int32)
            for r in range(NS):
                gacc = gacc + flags_v[r, pl.ds(0, 16)]
            return jnp.sum(gacc)

        gather_t()
        total = compute_pending()
        gtotal = publish(total)

        def round_body(_g):
            cps = [pltpu.async_copy(val2d.at[j], table.at[msk2d.at[j]], sem)
                   for j in range(NROW)]
            for cp in cps:
                cp.wait()
            plsc.subcore_barrier()
            gather_t()
            total = compute_pending()
            return publish(total)

        lax.while_loop(lambda g: g > 0, round_body, gtotal)

        pltpu.sync_copy(t_v, w_o.at[pl.ds(base, BPT)])

        for j in range(NROW):
            if j + 1 < NROW:
                c_cps.append(pltpu.async_copy(
                    c2_hbm.at[iq2d.at[j + 1]], stags[(j + 1) % 2], sem2))
            c_cps[j].wait()
            pltpu.sync_copy(stags[j % 2],
                            old_o.at[pl.ds(base + j * 128, 128)])

        for cp in cnt_cps:
            cp.wait()
        pltpu.sync_copy(cntf_v, cnt_o.at[pl.ds(base, BPT)])

        for r in range(NROW):
            for c in range(8):
                iq2d[r, pl.ds(c * 16, 16)] = lax.shift_right_logical(
                    t_v[pl.ds(r * 128 + c * 16, 16)], 2)
        x_cps = [pltpu.async_copy(x2_hbm.at[iq2d.at[0]], stags[0], sem2)]
        for j in range(NROW):
            if j + 1 < NROW:
                x_cps.append(pltpu.async_copy(
                    x2_hbm.at[iq2d.at[j + 1]], stags[(j + 1) % 2], sem2))
            x_cps[j].wait()
            pltpu.sync_copy(stags[j % 2],
                            xw_o.at[pl.ds(base + j * 128, 128)])


@jax.jit
def _sc_gather(i2d, x2, c2, counts):
    mesh = plsc.VectorSubcoreMesh(core_axis_name="c", subcore_axis_name="s")
    cp = pltpu.CompilerParams(needs_layout_passes=False,
                              use_tc_tiling_on_sc=False)
    f = pl.kernel(
        _sc_body,
        out_type=(
            jax.ShapeDtypeStruct((B, 128), jnp.float32),
            jax.ShapeDtypeStruct((B, 128), jnp.float32),
            jax.ShapeDtypeStruct((B,), jnp.float32),
            jax.ShapeDtypeStruct((B,), jnp.int32),
        ),
        mesh=mesh,
        scratch_types=[
            pltpu.VMEM((NROW, 128), jnp.int32),
            pltpu.VMEM((NROW, 128), jnp.int32),
            pltpu.VMEM((NROW, 128), jnp.int32),
            pltpu.VMEM((NROW, 128), jnp.int32),
            pltpu.VMEM((BPT,), jnp.int32),
            pltpu.VMEM((BPT,), jnp.float32),
            pltpu.VMEM((128, 128), jnp.float32),
            pltpu.VMEM((128, 128), jnp.float32),
            pltpu.VMEM((16,), jnp.int32),
            pltpu.VMEM((NS, 16), jnp.int32),
            pltpu.VMEM_SHARED((TPAD,), jnp.int32),
            pltpu.VMEM_SHARED((NS, 16), jnp.int32),
            pltpu.SemaphoreType.DMA,
            pltpu.SemaphoreType.DMA,
        ],
        compiler_params=cp,
    )
    return f(i2d, x2, c2, counts)


def _combine_body(old_ref, xw_ref, cnt_ref, i_ref, w_ref, out_ref):
    old = old_ref[...]
    xw = xw_ref[...]
    cnt = cnt_ref[...]
    si = i_ref[...] & 3
    sw = w_ref[...] & 3
    old32 = old[:, 0:D]
    xw32 = xw[:, 0:D]
    for k in range(1, 4):
        old32 = jnp.where(si == k, old[:, k * D:(k + 1) * D], old32)
        xw32 = jnp.where(sw == k, xw[:, k * D:(k + 1) * D], xw32)
    c = 1.0 - jnp.exp(LOG_ALPHA * (cnt + 1.0))
    new_c = old32 - (1.0 - ALPHA) * (old32 - xw32)
    out_ref[...] = new_c / c


_CBLK = 2048


@jax.jit
def _tc_combine(old, xw, cnt2d, i2d, w2d):
    return pl.pallas_call(
        _combine_body,
        grid=(B // _CBLK,),
        in_specs=[
            pl.BlockSpec((_CBLK, 128), lambda g: (g, 0)),
            pl.BlockSpec((_CBLK, 128), lambda g: (g, 0)),
            pl.BlockSpec((_CBLK, 1), lambda g: (g, 0)),
            pl.BlockSpec((_CBLK, 1), lambda g: (g, 0)),
            pl.BlockSpec((_CBLK, 1), lambda g: (g, 0)),
        ],
        out_specs=pl.BlockSpec((_CBLK, D), lambda g: (g, 0)),
        out_shape=jax.ShapeDtypeStruct((B, D), jnp.float32),
    )(old, xw, cnt2d, i2d, w2d)


def kernel(i, x, centers, counts):
    i32 = i.astype(jnp.int32)
    i2d = i32.reshape(128, 128)
    c2 = centers.reshape(M // 4, 128)
    x2 = x.reshape(B // 4, 128)
    old, xw, cnt, w = _sc_gather(i2d, x2, c2, counts)
    return _tc_combine(old, xw, cnt.reshape(B, 1), i32.reshape(B, 1),
                       w.reshape(B, 1))


# speedup vs baseline: 1.0665x; 1.0665x over previous
import dataclasses
import functools
import math

import jax
import jax.numpy as jnp
from jax import lax
from jax.experimental import pallas as pl
from jax.experimental.pallas import tpu as pltpu
from jax.experimental.pallas import tpu_sc as plsc

ALPHA = 0.99
LOG_ALPHA = math.log(ALPHA)

M = 1000000
D = 32
B = 16384

NS = 16
BPT = B // NS
NROW = 8
NDUM = 64
TPAD = M + NDUM * 16


def _sc_body(i_hbm, x_hbm, c_hbm, n_hbm,
             old_o, xw_o, cnt_o,
             idx2d, val2d, msk2d, t_v, cntf_v, old_v,
             myflag_v, flags_v, table, flags_sp, sem, sem2):
    core = lax.axis_index("c")
    sub = lax.axis_index("s")

    @pl.when(core == 0)
    def _():
        base = sub * BPT

        pltpu.sync_copy(i_hbm.at[pl.ds(sub * NROW, NROW)], idx2d)

        hbm_copies = []
        for j in range(NROW):
            hbm_copies.append(pltpu.async_copy(
                c_hbm.at[idx2d.at[j]],
                old_v.at[pl.ds(j * 128, 128)], sem2))
            hbm_copies.append(pltpu.async_copy(
                n_hbm.at[idx2d.at[j]],
                cntf_v.at[pl.ds(j * 128, 128)], sem2))

        for r in range(NROW):
            rbase = base + r * 128
            for c in range(8):
                val2d[r, pl.ds(c * 16, 16)] = (
                    rbase + c * 16 + lax.iota(jnp.int32, 16))

        cps = [pltpu.async_copy(val2d.at[j], table.at[idx2d.at[j]], sem)
               for j in range(NROW)]
        for cp in cps:
            cp.wait()
        plsc.subcore_barrier()

        def gather_t():
            cps = [pltpu.async_copy(table.at[idx2d.at[j]],
                                    t_v.at[pl.ds(j * 128, 128)], sem)
                   for j in range(NROW)]
            for cp in cps:
                cp.wait()

        def compute_pending():
            acc = jnp.zeros((16,), jnp.int32)
            rowcnt = []
            for r in range(NROW):
                racc = jnp.zeros((16,), jnp.int32)
                for c in range(8):
                    sl = pl.ds(c * 16, 16)
                    t16 = t_v[pl.ds(r * 128 + c * 16, 16)]
                    v16 = val2d[r, sl]
                    i16 = idx2d[r, sl]
                    pend = t16 < v16
                    grp = (sub * NROW * 8 + r * 8 + c) % NDUM
                    dummy = M + grp * 16 + lax.iota(jnp.int32, 16)
                    msk2d[r, sl] = jnp.where(pend, i16, dummy)
                    racc = racc + jnp.where(pend, 1, 0).astype(jnp.int32)
                acc = acc + racc
                rowcnt.append(jnp.sum(racc))
            return jnp.sum(acc), rowcnt

        def publish(total):
            myflag_v[...] = jnp.full((16,), total, jnp.int32)
            pltpu.sync_copy(myflag_v, flags_sp.at[sub])
            plsc.subcore_barrier()
            pltpu.sync_copy(flags_sp, flags_v)
            plsc.subcore_barrier()
            gacc = jnp.zeros((16,), jnp.int32)
            for r in range(NS):
                gacc = gacc + flags_v[r, pl.ds(0, 16)]
            return jnp.sum(gacc)

        gather_t()
        total, rowcnt = compute_pending()
        gtotal = publish(total)

        def round_body(_g):
            cps = [pltpu.async_copy(val2d.at[j], table.at[msk2d.at[j]], sem)
                   for j in range(NROW)]
            for cp in cps:
                cp.wait()
            plsc.subcore_barrier()
            gather_t()
            total, _ = compute_pending()
            return publish(total)

        lax.while_loop(lambda g: g > 0, round_body, gtotal)

        for cp in hbm_copies:
            cp.wait()
        pltpu.sync_copy(old_v, old_o.at[pl.ds(base, BPT)])
        pltpu.sync_copy(cntf_v, cnt_o.at[pl.ds(base, BPT)])

        cps = [pltpu.async_copy(x_hbm.at[t_v.at[pl.ds(j * 128, 128)]],
                                old_v.at[pl.ds(j * 128, 128)], sem)
               for j in range(NROW)]
        for cp in cps:
            cp.wait()
        pltpu.sync_copy(old_v, xw_o.at[pl.ds(base, BPT)])


@jax.jit
def _sc_gather(i2d, x, centers, counts):
    mesh = plsc.VectorSubcoreMesh(core_axis_name="c", subcore_axis_name="s")
    cp = pltpu.CompilerParams(needs_layout_passes=False,
                              use_tc_tiling_on_sc=False)
    f = pl.kernel(
        _sc_body,
        out_type=(
            jax.ShapeDtypeStruct((B, D), jnp.float32),
            jax.ShapeDtypeStruct((B, D), jnp.float32),
            jax.ShapeDtypeStruct((B,), jnp.float32),
        ),
        mesh=mesh,
        scratch_types=[
            pltpu.VMEM((NROW, 128), jnp.int32),
            pltpu.VMEM((NROW, 128), jnp.int32),
            pltpu.VMEM((NROW, 128), jnp.int32),
            pltpu.VMEM((BPT,), jnp.int32),
            pltpu.VMEM((BPT,), jnp.float32),
            pltpu.VMEM((BPT, D), jnp.float32),
            pltpu.VMEM((16,), jnp.int32),
            pltpu.VMEM((NS, 16), jnp.int32),
            pltpu.VMEM_SHARED((TPAD,), jnp.int32),
            pltpu.VMEM_SHARED((NS, 16), jnp.int32),
            pltpu.SemaphoreType.DMA,
            pltpu.SemaphoreType.DMA,
        ],
        compiler_params=cp,
    )
    return f(i2d, x, centers, counts)


def _combine_body(old_ref, xw_ref, cnt_ref, out_ref):
    old = old_ref[...]
    xw = xw_ref[...]
    cnt = cnt_ref[...]
    c = 1.0 - jnp.exp(LOG_ALPHA * (cnt + 1.0))
    new_c = old - (1.0 - ALPHA) * (old - xw)
    out_ref[...] = new_c / c


_CBLK = 2048


@jax.jit
def _tc_combine(old, xw, cnt2d):
    return pl.pallas_call(
        _combine_body,
        grid=(B // _CBLK,),
        in_specs=[
            pl.BlockSpec((_CBLK, D), lambda g: (g, 0)),
            pl.BlockSpec((_CBLK, D), lambda g: (g, 0)),
            pl.BlockSpec((_CBLK, 1), lambda g: (g, 0)),
        ],
        out_specs=pl.BlockSpec((_CBLK, D), lambda g: (g, 0)),
        out_shape=jax.ShapeDtypeStruct((B, D), jnp.float32),
    )(old, xw, cnt2d)


def kernel(i, x, centers, counts):
    i2d = i.astype(jnp.int32).reshape(128, 128)
    old, xw, cnt = _sc_gather(i2d, x, centers, counts)
    return _tc_combine(old, xw, cnt.reshape(B, 1))
